# Initial kernel scaffold; baseline (speedup 1.0000x reference)
import jax
import jax.numpy as jnp
from jax.experimental import pallas as pl


def kernel(x, edge_index, W1, b1, W2, b2, Wmu, bmu, Wlv, blv):
    raise NotImplementedError("write your pallas kernel here")


# speedup vs baseline: 15.0538x; 15.0538x over previous
import functools

import jax
import jax.numpy as jnp
from jax import lax
from jax.experimental import pallas as pl
from jax.experimental.pallas import tpu as pltpu
from jax.experimental.pallas import tpu_sc as plsc

N = 10000
NP = 10240
NC, NS = 2, 16
NWORK = NC * NS
CHUNK = 128
CH = 80
EPT = CH * CHUNK
ETOT = NWORK * EPT
RPT = NP // NS


def _sc_mesh():
    return plsc.VectorSubcoreMesh(core_axis_name="c", subcore_axis_name="s",
                                  num_cores=NC, num_subcores=NS)


def _deg_body(dst_hbm, ones_hbm, zeros_hbm, out_hbm, dst_v, ones_v, acc, sem):
    c = lax.axis_index("c")
    s = lax.axis_index("s")
    lin = c * NS + s
    pltpu.sync_copy(dst_hbm.at[lin], dst_v)
    pltpu.sync_copy(ones_hbm, ones_v)
    pltpu.sync_copy(zeros_hbm, acc.at[pl.ds(s * RPT, RPT)])
    plsc.subcore_barrier()
    for ch in range(CH):
        pltpu.sync_copy(ones_v, acc.at[dst_v.at[ch]], add=True)
    plsc.subcore_barrier()
    pltpu.sync_copy(acc.at[pl.ds(s * RPT, RPT)],
                    out_hbm.at[c, pl.ds(s * RPT, RPT)])


_sc_cache = {}


def _deg_call(*args):
    if "deg" not in _sc_cache:
        _sc_cache["deg"] = pl.kernel(
            _deg_body,
            out_type=jax.ShapeDtypeStruct((NC, NP), jnp.float32),
            mesh=_sc_mesh(),
            compiler_params=pltpu.CompilerParams(use_tc_tiling_on_sc=False),
            scratch_types=[
                pltpu.VMEM((CH, CHUNK), jnp.int32),
                pltpu.VMEM((CHUNK,), jnp.float32),
                pltpu.VMEM_SHARED((NP,), jnp.float32),
                pltpu.SemaphoreType.DMA,
            ],
        )
    return _sc_cache["deg"](*args)


def _make_prop(d):
    def body(g_hbm, src_hbm, dst_hbm, zeros_hbm, out_hbm,
             src_v, dst_v, rows, acc, gsem):
        c = lax.axis_index("c")
        s = lax.axis_index("s")
        lin = c * NS + s
        pltpu.sync_copy(src_hbm.at[lin], src_v)
        pltpu.sync_copy(dst_hbm.at[lin], dst_v)
        pltpu.sync_copy(zeros_hbm, acc.at[pl.ds(s * RPT, RPT)])
        plsc.subcore_barrier()
        for ch in range(CH):
            pltpu.async_copy(g_hbm.at[src_v.at[ch]], rows, gsem).wait()
            pltpu.sync_copy(rows, acc.at[dst_v.at[ch]], add=True)
        plsc.subcore_barrier()
        pltpu.sync_copy(acc.at[pl.ds(s * RPT, RPT)],
                        out_hbm.at[c, pl.ds(s * RPT, RPT)])

    return pl.kernel(
        body,
        out_type=jax.ShapeDtypeStruct((NC, NP, d), jnp.float32),
        mesh=_sc_mesh(),
        compiler_params=pltpu.CompilerParams(use_tc_tiling_on_sc=False),
        scratch_types=[
            pltpu.VMEM((CH, CHUNK), jnp.int32),
            pltpu.VMEM((CH, CHUNK), jnp.int32),
            pltpu.VMEM((CHUNK, d), jnp.float32),
            pltpu.VMEM_SHARED((NP, d), jnp.float32),
            pltpu.SemaphoreType.DMA,
        ],
    )


def _prop(d, *args):
    key = ("prop", d)
    if key not in _sc_cache:
        _sc_cache[key] = _make_prop(d)
    return _sc_cache[key](*args)


_BLK = 1024
_GRID = NP // _BLK


def _tc1_body(deg0, deg1, x_ref, w_ref, dinv_ref, g_ref):
    i = pl.program_id(0)
    deg = deg0[...] + deg1[...] + 1.0
    rows = lax.broadcasted_iota(jnp.int32, (_BLK, 1), 0) + i * _BLK
    dinv = jnp.where(rows < N, lax.rsqrt(deg), 0.0)
    dinv_ref[...] = dinv
    g_ref[...] = dinv * jnp.dot(x_ref[...], w_ref[...],
                                preferred_element_type=jnp.float32)


def _tc_mid_body(acc0, acc1, g_ref, dinv_ref, b_ref, w_ref, out_ref):
    dinv = dinv_ref[...]
    h = jnp.maximum(dinv * (acc0[...] + acc1[...] + g_ref[...]) + b_ref[...],
                    0.0)
    out_ref[...] = dinv * jnp.dot(h, w_ref[...],
                                  preferred_element_type=jnp.float32)


def _tc_mu_body(acc0, acc1, g_ref, dinv_ref, b_ref, mu_ref):
    mu_ref[...] = (dinv_ref[...] * (acc0[...] + acc1[...] + g_ref[...])
                   + b_ref[...])


_DBLK = 400
_DGRID = N // _DBLK


def _tc_dec_body(mu_i, mu_j, out_ref):
    logits = lax.dot_general(mu_i[...], mu_j[...],
                             (((1,), (1,)), ((), ())),
                             preferred_element_type=jnp.float32)
    out_ref[...] = jax.nn.sigmoid(logits)


def _row_spec(d):
    return pl.BlockSpec((_BLK, d), lambda i: (i, 0))


def _full_spec(shape):
    return pl.BlockSpec(shape, lambda i: tuple(0 for _ in shape))


def _tc1(deg0, deg1, x_p, W1):
    return pl.pallas_call(
        _tc1_body,
        grid=(_GRID,),
        in_specs=[_row_spec(1), _row_spec(1), _row_spec(128),
                  _full_spec((128, 64))],
        out_specs=[_row_spec(1), _row_spec(64)],
        out_shape=[jax.ShapeDtypeStruct((NP, 1), jnp.float32),
                   jax.ShapeDtypeStruct((NP, 64), jnp.float32)],
    )(deg0, deg1, x_p, W1)


def _tc_mid(acc, g, dinv, b, W, dn):
    d = g.shape[1]
    return pl.pallas_call(
        _tc_mid_body,
        grid=(_GRID,),
        in_specs=[_row_spec(d), _row_spec(d), _row_spec(d), _row_spec(1),
                  _full_spec((1, d)), _full_spec((d, dn))],
        out_specs=_row_spec(dn),
        out_shape=jax.ShapeDtypeStruct((NP, dn), jnp.float32),
    )(acc[0], acc[1], g, dinv, b, W)


def _tc_mu(acc, g, dinv, b):
    d = g.shape[1]
    return pl.pallas_call(
        _tc_mu_body,
        grid=(_GRID,),
        in_specs=[_row_spec(d), _row_spec(d), _row_spec(d), _row_spec(1),
                  _full_spec((1, d))],
        out_specs=_row_spec(d),
        out_shape=jax.ShapeDtypeStruct((NP, d), jnp.float32),
    )(acc[0], acc[1], g, dinv, b)


def _tc_dec(mu):
    return pl.pallas_call(
        _tc_dec_body,
        grid=(_DGRID,),
        in_specs=[pl.BlockSpec((_DBLK, 16), lambda i: (i, 0)),
                  pl.BlockSpec((N, 16), lambda i: (0, 0))],
        out_specs=pl.BlockSpec((_DBLK, N), lambda i: (i, 0)),
        out_shape=jax.ShapeDtypeStruct((N, N), jnp.float32),
    )(mu, mu)


def kernel(x, edge_index, W1, b1, W2, b2, Wmu, bmu, Wlv, blv):
    del Wlv, blv
    src = edge_index[0]
    dst = edge_index[1]
    pad = ETOT - src.shape[0]
    fill = jnp.full((pad,), N, dtype=jnp.int32)
    src_p = jnp.concatenate([src, fill]).reshape(NWORK, CH, CHUNK)
    dst_p = jnp.concatenate([dst, fill]).reshape(NWORK, CH, CHUNK)
    x_p = jnp.concatenate(
        [x, jnp.zeros((NP - N, x.shape[1]), jnp.float32)], axis=0)

    ones128 = jnp.ones((CHUNK,), jnp.float32)
    zrow = jnp.zeros((RPT,), jnp.float32)
    z64 = jnp.zeros((RPT, 64), jnp.float32)
    z16 = jnp.zeros((RPT, 16), jnp.float32)

    deg = _deg_call(dst_p, ones128, zrow)
    deg0 = deg[0].reshape(NP, 1)
    deg1 = deg[1].reshape(NP, 1)

    dinv, g1 = _tc1(deg0, deg1, x_p, W1)
    acc1 = _prop(64, g1, src_p, dst_p, z64)
    g2 = _tc_mid(acc1, g1, dinv, b1.reshape(1, 64), W2, 64)
    acc2 = _prop(64, g2, src_p, dst_p, z64)
    g3 = _tc_mid(acc2, g2, dinv, b2.reshape(1, 64), Wmu, 16)
    acc3 = _prop(16, g3, src_p, dst_p, z16)
    mu = _tc_mu(acc3, g3, dinv, bmu.reshape(1, 16))
    return _tc_dec(mu)


# speedup vs baseline: 16.6698x; 1.1073x over previous
import functools

import jax
import jax.numpy as jnp
from jax import lax
from jax.experimental import pallas as pl
from jax.experimental.pallas import tpu as pltpu
from jax.experimental.pallas import tpu_sc as plsc

N = 10000
NP = 10240
NC, NS = 2, 16
NWORK = NC * NS
CHUNK = 128
CH = 80
EPT = CH * CHUNK
ETOT = NWORK * EPT
RPT = NP // NS


def _sc_mesh():
    return plsc.VectorSubcoreMesh(core_axis_name="c", subcore_axis_name="s",
                                  num_cores=NC, num_subcores=NS)


def _deg_body(dst_hbm, ones_hbm, zeros_hbm, out_hbm, dst_v, ones_v, acc,
              *sems):
    c = lax.axis_index("c")
    s = lax.axis_index("s")
    lin = c * NS + s
    pltpu.sync_copy(dst_hbm.at[lin], dst_v)
    pltpu.sync_copy(ones_hbm, ones_v)
    pltpu.sync_copy(zeros_hbm, acc.at[pl.ds(s * RPT, RPT)])
    plsc.subcore_barrier()
    dsc = {}
    for ch in range(CH):
        if ch >= 4:
            dsc[ch - 4].wait()
        dsc[ch] = pltpu.async_copy(ones_v, acc.at[dst_v.at[ch]],
                                   sems[ch % 4], add=True)
    for ch in range(CH - 4, CH):
        dsc[ch].wait()
    plsc.subcore_barrier()
    pltpu.sync_copy(acc.at[pl.ds(s * RPT, RPT)],
                    out_hbm.at[c, pl.ds(s * RPT, RPT)])


_sc_cache = {}


def _deg_call(*args):
    if "deg" not in _sc_cache:
        _sc_cache["deg"] = pl.kernel(
            _deg_body,
            out_type=jax.ShapeDtypeStruct((NC, NP), jnp.float32),
            mesh=_sc_mesh(),
            compiler_params=pltpu.CompilerParams(use_tc_tiling_on_sc=False),
            scratch_types=[
                pltpu.VMEM((CH, CHUNK), jnp.int32),
                pltpu.VMEM((CHUNK,), jnp.float32),
                pltpu.VMEM_SHARED((NP,), jnp.float32),
            ] + [pltpu.SemaphoreType.DMA] * 4,
        )
    return _sc_cache["deg"](*args)


NBUF = 8
LAG = 4


def _make_prop(d):
    def body(g_hbm, src_hbm, dst_hbm, zeros_hbm, out_hbm,
             src_v, dst_v, rows, acc, *sems):
        gsem = sems[:NBUF]
        ssem = sems[NBUF:]
        c = lax.axis_index("c")
        s = lax.axis_index("s")
        lin = c * NS + s
        pltpu.sync_copy(src_hbm.at[lin], src_v)
        pltpu.sync_copy(dst_hbm.at[lin], dst_v)
        pltpu.sync_copy(zeros_hbm, acc.at[pl.ds(s * RPT, RPT)])
        plsc.subcore_barrier()
        dg = {}
        dsc = {}
        for j in range(LAG):
            dg[j] = pltpu.async_copy(g_hbm.at[src_v.at[j]],
                                     rows.at[j % NBUF], gsem[j % NBUF])
        for ch in range(CH):
            b = ch % NBUF
            dg[ch].wait()
            dsc[ch] = pltpu.async_copy(rows.at[b], acc.at[dst_v.at[ch]],
                                       ssem[b], add=True)
            g = ch + LAG
            if g < CH:
                bg = g % NBUF
                if g - NBUF >= 0:
                    dsc[g - NBUF].wait()
                dg[g] = pltpu.async_copy(g_hbm.at[src_v.at[g]],
                                         rows.at[bg], gsem[bg])
        for ch in range(CH - NBUF, CH):
            dsc[ch].wait()
        plsc.subcore_barrier()
        pltpu.sync_copy(acc.at[pl.ds(s * RPT, RPT)],
                        out_hbm.at[c, pl.ds(s * RPT, RPT)])

    return pl.kernel(
        body,
        out_type=jax.ShapeDtypeStruct((NC, NP, d), jnp.float32),
        mesh=_sc_mesh(),
        compiler_params=pltpu.CompilerParams(use_tc_tiling_on_sc=False),
        scratch_types=[
            pltpu.VMEM((CH, CHUNK), jnp.int32),
            pltpu.VMEM((CH, CHUNK), jnp.int32),
            pltpu.VMEM((NBUF, CHUNK, d), jnp.float32),
            pltpu.VMEM_SHARED((NP, d), jnp.float32),
        ] + [pltpu.SemaphoreType.DMA] * (2 * NBUF),
    )


def _prop(d, *args):
    key = ("prop", d)
    if key not in _sc_cache:
        _sc_cache[key] = _make_prop(d)
    return _sc_cache[key](*args)


_BLK = 1024
_GRID = NP // _BLK


def _tc1_body(deg0, deg1, x_ref, w_ref, dinv_ref, g_ref):
    i = pl.program_id(0)
    deg = deg0[...] + deg1[...] + 1.0
    rows = lax.broadcasted_iota(jnp.int32, (_BLK, 1), 0) + i * _BLK
    dinv = jnp.where(rows < N, lax.rsqrt(deg), 0.0)
    dinv_ref[...] = dinv
    g_ref[...] = dinv * jnp.dot(x_ref[...], w_ref[...],
                                preferred_element_type=jnp.float32)


def _tc_mid_body(acc0, acc1, g_ref, dinv_ref, b_ref, w_ref, out_ref):
    dinv = dinv_ref[...]
    h = jnp.maximum(dinv * (acc0[...] + acc1[...] + g_ref[...]) + b_ref[...],
                    0.0)
    out_ref[...] = dinv * jnp.dot(h, w_ref[...],
                                  preferred_element_type=jnp.float32)


def _tc_mu_body(acc0, acc1, g_ref, dinv_ref, b_ref, mu_ref):
    mu_ref[...] = (dinv_ref[...] * (acc0[...] + acc1[...] + g_ref[...])
                   + b_ref[...])


_DBLK = 400
_DGRID = N // _DBLK


def _tc_dec_body(mu_i, mu_j, out_ref):
    logits = lax.dot_general(mu_i[...], mu_j[...],
                             (((1,), (1,)), ((), ())),
                             preferred_element_type=jnp.float32)
    out_ref[...] = jax.nn.sigmoid(logits)


def _row_spec(d):
    return pl.BlockSpec((_BLK, d), lambda i: (i, 0))


def _full_spec(shape):
    return pl.BlockSpec(shape, lambda i: tuple(0 for _ in shape))


def _tc1(deg0, deg1, x_p, W1):
    return pl.pallas_call(
        _tc1_body,
        grid=(_GRID,),
        in_specs=[_row_spec(1), _row_spec(1), _row_spec(128),
                  _full_spec((128, 64))],
        out_specs=[_row_spec(1), _row_spec(64)],
        out_shape=[jax.ShapeDtypeStruct((NP, 1), jnp.float32),
                   jax.ShapeDtypeStruct((NP, 64), jnp.float32)],
    )(deg0, deg1, x_p, W1)


def _tc_mid(acc, g, dinv, b, W, dn):
    d = g.shape[1]
    return pl.pallas_call(
        _tc_mid_body,
        grid=(_GRID,),
        in_specs=[_row_spec(d), _row_spec(d), _row_spec(d), _row_spec(1),
                  _full_spec((1, d)), _full_spec((d, dn))],
        out_specs=_row_spec(dn),
        out_shape=jax.ShapeDtypeStruct((NP, dn), jnp.float32),
    )(acc[0], acc[1], g, dinv, b, W)


def _tc_mu(acc, g, dinv, b):
    d = g.shape[1]
    return pl.pallas_call(
        _tc_mu_body,
        grid=(_GRID,),
        in_specs=[_row_spec(d), _row_spec(d), _row_spec(d), _row_spec(1),
                  _full_spec((1, d))],
        out_specs=_row_spec(d),
        out_shape=jax.ShapeDtypeStruct((NP, d), jnp.float32),
    )(acc[0], acc[1], g, dinv, b)


def _tc_dec(mu):
    return pl.pallas_call(
        _tc_dec_body,
        grid=(_DGRID,),
        in_specs=[pl.BlockSpec((_DBLK, 16), lambda i: (i, 0)),
                  pl.BlockSpec((N, 16), lambda i: (0, 0))],
        out_specs=pl.BlockSpec((_DBLK, N), lambda i: (i, 0)),
        out_shape=jax.ShapeDtypeStruct((N, N), jnp.float32),
    )(mu, mu)


def kernel(x, edge_index, W1, b1, W2, b2, Wmu, bmu, Wlv, blv):
    del Wlv, blv
    src = edge_index[0]
    dst = edge_index[1]
    pad = ETOT - src.shape[0]
    fill = jnp.full((pad,), N, dtype=jnp.int32)
    src_p = jnp.concatenate([src, fill]).reshape(NWORK, CH, CHUNK)
    dst_p = jnp.concatenate([dst, fill]).reshape(NWORK, CH, CHUNK)
    x_p = jnp.concatenate(
        [x, jnp.zeros((NP - N, x.shape[1]), jnp.float32)], axis=0)

    ones128 = jnp.ones((CHUNK,), jnp.float32)
    zrow = jnp.zeros((RPT,), jnp.float32)
    z64 = jnp.zeros((RPT, 64), jnp.float32)
    z16 = jnp.zeros((RPT, 16), jnp.float32)

    deg = _deg_call(dst_p, ones128, zrow)
    deg0 = deg[0].reshape(NP, 1)
    deg1 = deg[1].reshape(NP, 1)

    dinv, g1 = _tc1(deg0, deg1, x_p, W1)
    acc1 = _prop(64, g1, src_p, dst_p, z64)
    g2 = _tc_mid(acc1, g1, dinv, b1.reshape(1, 64), W2, 64)
    acc2 = _prop(64, g2, src_p, dst_p, z64)
    g3 = _tc_mid(acc2, g2, dinv, b2.reshape(1, 64), Wmu, 16)
    acc3 = _prop(16, g3, src_p, dst_p, z16)
    mu = _tc_mu(acc3, g3, dinv, bmu.reshape(1, 16))
    return _tc_dec(mu)
